# baseline (speedup 1.0000x reference)
import numpy as np

import jax
import jax.numpy as jnp
from jax import lax
from jax.experimental import pallas as pl
from jax.experimental.pallas import tpu as pltpu
from jax.experimental.pallas import tpu_sc as plsc

N_NODE = 10000
E = 320000
D = 128
DP = 144
NROWS = 10112
JUNK = 10000
NC, NS = 2, 16
RPT = NROWS // NS
K = 64
C1 = 320
C2 = 160
BMM = 2000
BM = 1000

_PERM = np.empty((D,), np.int32)
for _g in range(4):
    for _i in range(16):
        _PERM[32 * _g + 2 * _i] = 32 * _g + _i
        _PERM[32 * _g + 2 * _i + 1] = 32 * _g + 16 + _i
_UNPERM = np.zeros((D, D), np.float32)
_UNPERM[_PERM, np.arange(D)] = 1.0


def _mm_body(xu, xi, wf, wc, wcb, bf, bc, bcb, of, oc, ocb):
    of[...] = (jnp.dot(xu[...], wf[...], preferred_element_type=jnp.float32)
               + bf[...]).astype(jnp.bfloat16)
    oc[...] = (jnp.dot(xu[...], wc[...], preferred_element_type=jnp.float32)
               + bc[...]).astype(jnp.bfloat16)
    ocb[...] = (jnp.dot(xi[...], wcb[...], preferred_element_type=jnp.float32)
                + bcb[...]).astype(jnp.bfloat16)


def _make_tables(x_user, x_item, W_f, b_f, W_c, b_c, W_cb, b_cb):
    grid = N_NODE // BMM
    full_w = pl.BlockSpec((D, D), lambda i: (0, 0))
    full_b = pl.BlockSpec((1, D), lambda i: (0, 0))
    row_blk = pl.BlockSpec((BMM, D), lambda i: (i, 0))
    out_sds = jax.ShapeDtypeStruct((N_NODE, D), jnp.bfloat16)
    return pl.pallas_call(
        _mm_body,
        grid=(grid,),
        in_specs=[row_blk, row_blk, full_w, full_w, full_w,
                  full_b, full_b, full_b],
        out_specs=(row_blk, row_blk, row_blk),
        out_shape=(out_sds, out_sds, out_sds),
    )(x_user, x_item, W_f, W_c, W_cb, b_f.reshape(1, D), b_c.reshape(1, D),
      b_cb.reshape(1, D))


def _sc_body(whp_f, whp_c, whp_cb, s1, d1, s2, d2, zeros_hbm, finit_hbm,
             out1, out2, s_all, d_all, rows, frows, acc, *sem):
    cc = lax.axis_index("c")
    tid = lax.axis_index("s")
    semg = sem[0:4]
    sems = sem[4:6]
    semi = sem[6:14]

    pltpu.sync_copy(finit_hbm, frows.at[0])
    pltpu.sync_copy(finit_hbm, frows.at[1])


    def run_phase(table, s_e, d_e, nchunks, out):
        pltpu.sync_copy(zeros_hbm, acc.at[pl.ds(tid * RPT, RPT)])
        plsc.subcore_barrier()

        def stage_idx(ib, j):
            pltpu.async_copy(s_e.at[cc, tid, j], s_all.at[ib], semi[ib])
            pltpu.async_copy(d_e.at[cc, tid, j], d_all.at[ib], semi[ib])

        def wait_idx(ib, j):
            pltpu.make_async_copy(s_e.at[cc, tid, j], s_all.at[ib],
                                  semi[ib]).wait()
            pltpu.make_async_copy(d_e.at[cc, tid, j], d_all.at[ib],
                                  semi[ib]).wait()

        def start_gather(grb, ib, j):
            wait_idx(ib, j)
            pltpu.async_copy(table.at[s_all.at[ib]], rows.at[grb], semg[grb])

        def wait_gather(grb):
            pltpu.make_async_copy(table.at[s_all.at[0]], rows.at[grb],
                                  semg[grb]).wait()

        def start_scatter(fb, ib):
            pltpu.async_copy(frows.at[fb], acc.at[d_all.at[ib]], sems[fb],
                             add=True)

        def wait_scatter(fb):
            pltpu.make_async_copy(frows.at[fb], acc.at[d_all.at[0]],
                                  sems[fb]).wait()

        def convert(grb, fb):
            @plsc.parallel_loop(0, K, unroll=8)
            def conv_row(r):
                for g in range(4):
                    x = rows[grb, r, pl.ds(g * 32, 32)]
                    a, b = plsc.unpack(x, format=plsc.PackFormat.INTERLEAVED)
                    frows[fb, r, pl.ds(g * 32, 16)] = a
                    frows[fb, r, pl.ds(g * 32 + 16, 16)] = b

        stage_idx(0, 0)
        stage_idx(1, 1)

        def body8(jj, carry):
            for b in range(8):
                j = jj * 8 + b
                grb = b % 4
                fb = b % 2

                @pl.when(j >= 6)
                def _():
                    wait_scatter(fb)

                @pl.when(j >= 4)
                def _():
                    wait_gather(grb)
                    convert(grb, fb)
                    start_scatter(fb, (b + 4) % 8)

                @pl.when(j + 2 < nchunks)
                def _():
                    stage_idx((b + 2) % 8, j + 2)

                start_gather(grb, b, j)
            return carry

        lax.fori_loop(0, nchunks // 8, body8, 0)
        for t in range(4):
            wait_scatter(t % 2)
            wait_gather(t)
            convert(t, t % 2)
            start_scatter(t % 2, 4 + t)
        wait_scatter(0)
        wait_scatter(1)
        plsc.subcore_barrier()
        pltpu.sync_copy(acc.at[pl.ds(tid * RPT, RPT)],
                        out.at[cc, pl.ds(tid * RPT, RPT)])

    @pl.when(cc == 0)
    def _():
        run_phase(whp_f, s1, d1, C1, out1)

    @pl.when(cc == 1)
    def _():
        run_phase(whp_cb, s1, d1, C1, out1)

    run_phase(whp_c, s2, d2, C2, out2)


def _comb_body(o1, o2, unperm, hu, hi):
    sf = o1[0, :, :D]
    cf = o1[0, :, D:D + 1]
    scb = o1[1, :, :D]
    ccb = o1[1, :, D:D + 1]
    tu = sf / jnp.maximum(cf, 1.0) + scb / jnp.maximum(ccb, 1.0)
    hu[...] = jnp.dot(tu, unperm[...], preferred_element_type=jnp.float32)
    s0 = o2[0, :, :D]
    c0 = o2[0, :, D:D + 1]
    s1_ = o2[1, :, :D]
    c1 = o2[1, :, D:D + 1]
    ti = (s0 + s1_) / jnp.maximum(c0 + c1, 1.0)
    hi[...] = jnp.dot(ti, unperm[...], preferred_element_type=jnp.float32)


def _pad_edges(idx, fill, per_tile_chunks):
    total = NS * per_tile_chunks * K
    out = jnp.full((total,), fill, dtype=jnp.int32).at[: idx.shape[0]].set(idx)
    return out.reshape(NS, per_tile_chunks, K)


def kernel(x_user, x_item, edge_follows, edge_clicks, edge_clicked_by,
           W_follows, b_follows, W_clicks, b_clicks, W_clicked_by,
           b_clicked_by):
    whp_f, whp_c, whp_cb = _make_tables(
        x_user, x_item, W_follows, b_follows, W_clicks, b_clicks,
        W_clicked_by, b_clicked_by)

    s1 = jnp.stack([_pad_edges(edge_follows[0], 0, C1),
                    _pad_edges(edge_clicked_by[0], 0, C1)])
    d1 = jnp.stack([_pad_edges(edge_follows[1], JUNK, C1),
                    _pad_edges(edge_clicked_by[1], JUNK, C1)])
    half = E // 2
    s2 = jnp.stack([_pad_edges(edge_clicks[0, :half], 0, C2),
                    _pad_edges(edge_clicks[0, half:], 0, C2)])
    d2 = jnp.stack([_pad_edges(edge_clicks[1, :half], JUNK, C2),
                    _pad_edges(edge_clicks[1, half:], JUNK, C2)])
    zeros_hbm = jnp.zeros((RPT, DP), jnp.float32)
    finit_hbm = jnp.zeros((K, DP), jnp.float32).at[:, D].set(1.0)

    sc = pl.kernel(
        _sc_body,
        out_type=(jax.ShapeDtypeStruct((NC, NROWS, DP), jnp.float32),
                  jax.ShapeDtypeStruct((NC, NROWS, DP), jnp.float32)),
        mesh=plsc.VectorSubcoreMesh(core_axis_name="c", subcore_axis_name="s"),
        scratch_types=[
            pltpu.VMEM((8, K), jnp.int32),
            pltpu.VMEM((8, K), jnp.int32),
            pltpu.VMEM((4, K, D), jnp.bfloat16),
            pltpu.VMEM((2, K, DP), jnp.float32),
            pltpu.VMEM_SHARED((NROWS, DP), jnp.float32),
        ] + [pltpu.SemaphoreType.DMA] * 14,
        compiler_params=pltpu.CompilerParams(use_tc_tiling_on_sc=False,
                                            needs_layout_passes=False),
    )
    out1, out2 = sc(whp_f, whp_c, whp_cb, s1, d1, s2, d2, zeros_hbm, finit_hbm)

    grid = N_NODE // BM
    h_user, h_item = pl.pallas_call(
        _comb_body,
        grid=(grid,),
        in_specs=[pl.BlockSpec((NC, BM, DP), lambda i: (0, i, 0)),
                  pl.BlockSpec((NC, BM, DP), lambda i: (0, i, 0)),
                  pl.BlockSpec((D, D), lambda i: (0, 0))],
        out_specs=(pl.BlockSpec((BM, D), lambda i: (i, 0)),
                   pl.BlockSpec((BM, D), lambda i: (i, 0))),
        out_shape=(jax.ShapeDtypeStruct((N_NODE, D), jnp.float32),
                   jax.ShapeDtypeStruct((N_NODE, D), jnp.float32)),
    )(out1, out2, jnp.asarray(_UNPERM))
    return (h_user, h_item)


# speedup vs baseline: 1.0062x; 1.0062x over previous
import numpy as np

import jax
import jax.numpy as jnp
from jax import lax
from jax.experimental import pallas as pl
from jax.experimental.pallas import tpu as pltpu
from jax.experimental.pallas import tpu_sc as plsc

N_NODE = 10000
E = 320000
D = 128
DP = 144
NROWS = 10112
JUNK = 10000
NC, NS = 2, 16
RPT = NROWS // NS
K = 64
C1 = 320
C2 = 160
BMM = 2000
BM = 1000

_PERM = np.empty((D,), np.int32)
for _g in range(4):
    for _i in range(16):
        _PERM[32 * _g + 2 * _i] = 32 * _g + _i
        _PERM[32 * _g + 2 * _i + 1] = 32 * _g + 16 + _i
_UNPERM = np.zeros((D, D), np.float32)
_UNPERM[_PERM, np.arange(D)] = 1.0


def _mm_body(xu, xi, wf, wc, wcb, bf, bc, bcb, of, oc, ocb):
    of[...] = (jnp.dot(xu[...], wf[...], preferred_element_type=jnp.float32)
               + bf[...]).astype(jnp.bfloat16)
    oc[...] = (jnp.dot(xu[...], wc[...], preferred_element_type=jnp.float32)
               + bc[...]).astype(jnp.bfloat16)
    ocb[...] = (jnp.dot(xi[...], wcb[...], preferred_element_type=jnp.float32)
                + bcb[...]).astype(jnp.bfloat16)


def _make_tables(x_user, x_item, W_f, b_f, W_c, b_c, W_cb, b_cb):
    grid = N_NODE // BMM
    full_w = pl.BlockSpec((D, D), lambda i: (0, 0))
    full_b = pl.BlockSpec((1, D), lambda i: (0, 0))
    row_blk = pl.BlockSpec((BMM, D), lambda i: (i, 0))
    out_sds = jax.ShapeDtypeStruct((N_NODE, D), jnp.bfloat16)
    return pl.pallas_call(
        _mm_body,
        grid=(grid,),
        in_specs=[row_blk, row_blk, full_w, full_w, full_w,
                  full_b, full_b, full_b],
        out_specs=(row_blk, row_blk, row_blk),
        out_shape=(out_sds, out_sds, out_sds),
    )(x_user, x_item, W_f, W_c, W_cb, b_f.reshape(1, D), b_c.reshape(1, D),
      b_cb.reshape(1, D))


def _sc_body(whp_f, whp_c, whp_cb, s1, d1, s2, d2, zeros_hbm, finit_hbm,
             out1, out2, s_all, d_all, rows, frows, acc, zbuf, *sem):
    cc = lax.axis_index("c")
    tid = lax.axis_index("s")
    semg = sem[0:4]
    sems = sem[4:6]
    semi = sem[6:14]

    pltpu.sync_copy(finit_hbm, frows.at[0])
    pltpu.sync_copy(finit_hbm, frows.at[1])
    pltpu.sync_copy(zeros_hbm, zbuf)


    def run_phase(table, s_e, d_e, nchunks, out):
        def zero_blk(r, carry):
            pltpu.async_copy(zbuf, acc.at[pl.ds(tid * RPT + r * 8, 8)],
                             sem[14])
            return carry

        lax.fori_loop(0, RPT // 8, zero_blk, 0)

        def zero_drain(r, carry):
            pltpu.make_async_copy(zbuf, acc.at[pl.ds(tid * RPT, 8)],
                                  sem[14]).wait()
            return carry

        lax.fori_loop(0, RPT // 8, zero_drain, 0)
        plsc.subcore_barrier()

        def stage_idx(ib, j):
            pltpu.async_copy(s_e.at[cc, tid, j], s_all.at[ib], semi[ib])
            pltpu.async_copy(d_e.at[cc, tid, j], d_all.at[ib], semi[ib])

        def wait_idx(ib, j):
            pltpu.make_async_copy(s_e.at[cc, tid, j], s_all.at[ib],
                                  semi[ib]).wait()
            pltpu.make_async_copy(d_e.at[cc, tid, j], d_all.at[ib],
                                  semi[ib]).wait()

        def start_gather(grb, ib, j):
            wait_idx(ib, j)
            pltpu.async_copy(table.at[s_all.at[ib]], rows.at[grb], semg[grb])

        def wait_gather(grb):
            pltpu.make_async_copy(table.at[s_all.at[0]], rows.at[grb],
                                  semg[grb]).wait()

        def start_scatter(fb, ib):
            pltpu.async_copy(frows.at[fb], acc.at[d_all.at[ib]], sems[fb],
                             add=True)

        def wait_scatter(fb):
            pltpu.make_async_copy(frows.at[fb], acc.at[d_all.at[0]],
                                  sems[fb]).wait()

        def convert(grb, fb):
            @plsc.parallel_loop(0, K, unroll=8)
            def conv_row(r):
                for g in range(4):
                    x = rows[grb, r, pl.ds(g * 32, 32)]
                    a, b = plsc.unpack(x, format=plsc.PackFormat.INTERLEAVED)
                    frows[fb, r, pl.ds(g * 32, 16)] = a
                    frows[fb, r, pl.ds(g * 32 + 16, 16)] = b

        stage_idx(0, 0)
        stage_idx(1, 1)

        def body8(jj, carry):
            for b in range(8):
                j = jj * 8 + b
                grb = b % 4
                fb = b % 2

                @pl.when(j >= 6)
                def _():
                    wait_scatter(fb)

                @pl.when(j >= 4)
                def _():
                    wait_gather(grb)
                    convert(grb, fb)
                    start_scatter(fb, (b + 4) % 8)

                @pl.when(j + 2 < nchunks)
                def _():
                    stage_idx((b + 2) % 8, j + 2)

                start_gather(grb, b, j)
            return carry

        lax.fori_loop(0, nchunks // 8, body8, 0)
        for t in range(4):
            wait_scatter(t % 2)
            wait_gather(t)
            convert(t, t % 2)
            start_scatter(t % 2, 4 + t)
        wait_scatter(0)
        wait_scatter(1)
        plsc.subcore_barrier()
        pltpu.sync_copy(acc.at[pl.ds(tid * RPT, RPT)],
                        out.at[cc, pl.ds(tid * RPT, RPT)])

    @pl.when(cc == 0)
    def _():
        run_phase(whp_f, s1, d1, C1, out1)

    @pl.when(cc == 1)
    def _():
        run_phase(whp_cb, s1, d1, C1, out1)

    run_phase(whp_c, s2, d2, C2, out2)


def _comb_body(o1, o2, unperm, hu, hi):
    sf = o1[0, :, :D]
    cf = o1[0, :, D:D + 1]
    scb = o1[1, :, :D]
    ccb = o1[1, :, D:D + 1]
    tu = sf / jnp.maximum(cf, 1.0) + scb / jnp.maximum(ccb, 1.0)
    hu[...] = jnp.dot(tu, unperm[...], preferred_element_type=jnp.float32)
    s0 = o2[0, :, :D]
    c0 = o2[0, :, D:D + 1]
    s1_ = o2[1, :, :D]
    c1 = o2[1, :, D:D + 1]
    ti = (s0 + s1_) / jnp.maximum(c0 + c1, 1.0)
    hi[...] = jnp.dot(ti, unperm[...], preferred_element_type=jnp.float32)


def _pad_edges(idx, fill, per_tile_chunks):
    total = NS * per_tile_chunks * K
    out = jnp.full((total,), fill, dtype=jnp.int32).at[: idx.shape[0]].set(idx)
    return out.reshape(NS, per_tile_chunks, K)


def kernel(x_user, x_item, edge_follows, edge_clicks, edge_clicked_by,
           W_follows, b_follows, W_clicks, b_clicks, W_clicked_by,
           b_clicked_by):
    whp_f, whp_c, whp_cb = _make_tables(
        x_user, x_item, W_follows, b_follows, W_clicks, b_clicks,
        W_clicked_by, b_clicked_by)

    s1 = jnp.stack([_pad_edges(edge_follows[0], 0, C1),
                    _pad_edges(edge_clicked_by[0], 0, C1)])
    d1 = jnp.stack([_pad_edges(edge_follows[1], JUNK, C1),
                    _pad_edges(edge_clicked_by[1], JUNK, C1)])
    half = E // 2
    s2 = jnp.stack([_pad_edges(edge_clicks[0, :half], 0, C2),
                    _pad_edges(edge_clicks[0, half:], 0, C2)])
    d2 = jnp.stack([_pad_edges(edge_clicks[1, :half], JUNK, C2),
                    _pad_edges(edge_clicks[1, half:], JUNK, C2)])
    zeros_hbm = jnp.zeros((8, DP), jnp.float32)
    finit_hbm = jnp.zeros((K, DP), jnp.float32).at[:, D].set(1.0)

    sc = pl.kernel(
        _sc_body,
        out_type=(jax.ShapeDtypeStruct((NC, NROWS, DP), jnp.float32),
                  jax.ShapeDtypeStruct((NC, NROWS, DP), jnp.float32)),
        mesh=plsc.VectorSubcoreMesh(core_axis_name="c", subcore_axis_name="s"),
        scratch_types=[
            pltpu.VMEM((8, K), jnp.int32),
            pltpu.VMEM((8, K), jnp.int32),
            pltpu.VMEM((4, K, D), jnp.bfloat16),
            pltpu.VMEM((2, K, DP), jnp.float32),
            pltpu.VMEM_SHARED((NROWS, DP), jnp.float32),
            pltpu.VMEM((8, DP), jnp.float32),
        ] + [pltpu.SemaphoreType.DMA] * 15,
        compiler_params=pltpu.CompilerParams(use_tc_tiling_on_sc=False,
                                            needs_layout_passes=False),
    )
    out1, out2 = sc(whp_f, whp_c, whp_cb, s1, d1, s2, d2, zeros_hbm, finit_hbm)

    grid = N_NODE // BM
    h_user, h_item = pl.pallas_call(
        _comb_body,
        grid=(grid,),
        in_specs=[pl.BlockSpec((NC, BM, DP), lambda i: (0, i, 0)),
                  pl.BlockSpec((NC, BM, DP), lambda i: (0, i, 0)),
                  pl.BlockSpec((D, D), lambda i: (0, 0))],
        out_specs=(pl.BlockSpec((BM, D), lambda i: (i, 0)),
                   pl.BlockSpec((BM, D), lambda i: (i, 0))),
        out_shape=(jax.ShapeDtypeStruct((N_NODE, D), jnp.float32),
                   jax.ShapeDtypeStruct((N_NODE, D), jnp.float32)),
    )(out1, out2, jnp.asarray(_UNPERM))
    return (h_user, h_item)
